# baseline (speedup 1.0000x reference)
import functools

import jax
import jax.numpy as jnp
from jax import lax
from jax.experimental import pallas as pl
from jax.experimental.pallas import tpu as pltpu
from jax.experimental.pallas import tpu_sc as plsc

N = 10000
DEG = 16
E = N * DEG
D = 128
N_CLS = 40
D_OUT_PAD = 48

NC = 2
NS = 16
NW = NC * NS

G_PERW = E // NW
G_CHUNK = 200
G_NCH = G_PERW // G_CHUNK

NP = 10240
R_PERW = NP // NW
R_CHUNK = 32
R_NCH = R_PERW // R_CHUNK
EP = NP * DEG


def _sc_mesh():
  return plsc.VectorSubcoreMesh(
      core_axis_name="c", subcore_axis_name="s",
      num_cores=NC, num_subcores=NS)


def _gather_rows(table, idx, n_rows, width, chunk, untiled=False,
                 dtype=jnp.float32):
  perw = n_rows // NW
  nch = perw // chunk
  params = (pltpu.CompilerParams(use_tc_tiling_on_sc=False)
            if untiled else None)

  @functools.partial(
      pl.kernel,
      out_type=jax.ShapeDtypeStruct((n_rows, width), dtype),
      mesh=_sc_mesh(),
      scratch_types=[
          pltpu.VMEM((chunk,), jnp.int32),
          pltpu.VMEM((chunk,), jnp.int32),
          pltpu.VMEM((chunk, width), dtype),
          pltpu.VMEM((chunk, width), dtype),
          pltpu.SemaphoreType.DMA,
          pltpu.SemaphoreType.DMA,
          pltpu.SemaphoreType.DMA,
          pltpu.SemaphoreType.DMA,
      ],
      compiler_params=params,
  )
  def k(table_hbm, idx_hbm, out_hbm, idx0, idx1, rows0, rows1, g0, g1, s0, s1):
    G_CHUNK = chunk
    G_NCH = nch
    wid = lax.axis_index("s") * NC + lax.axis_index("c")
    base = wid * perw
    idx_v = (idx0, idx1)
    rows_v = (rows0, rows1)
    gsem = (g0, g1)
    ssem = (s0, s1)
    gathers = [None, None]
    scatters = [None, None]
    for j in range(G_NCH):
      sl = j & 1
      off = base + j * G_CHUNK
      if scatters[sl] is not None:
        scatters[sl].wait()
        scatters[sl] = None
      pltpu.sync_copy(idx_hbm.at[pl.ds(off, G_CHUNK)], idx_v[sl])
      gathers[sl] = pltpu.async_copy(
          table_hbm.at[idx_v[sl]], rows_v[sl], gsem[sl])
      po = 1 - sl
      if gathers[po] is not None:
        gathers[po].wait()
        poff = base + (j - 1) * G_CHUNK
        scatters[po] = pltpu.async_copy(
            rows_v[po], out_hbm.at[pl.ds(poff, G_CHUNK)], ssem[po])
        gathers[po] = None
    sl = (G_NCH - 1) & 1
    gathers[sl].wait()
    if scatters[sl] is not None:
      scatters[sl].wait()
    off = base + (G_NCH - 1) * G_CHUNK
    pltpu.sync_copy(rows_v[sl], out_hbm.at[pl.ds(off, G_CHUNK)])
    scatters[1 - sl].wait()

  return k(table, idx)


def _gather_reduce(table, idx):
  rows_per_chunk = R_CHUNK * DEG

  @functools.partial(
      pl.kernel,
      out_type=jax.ShapeDtypeStruct((NP, D_OUT_PAD), jnp.float32),
      mesh=_sc_mesh(),
      scratch_types=[
          pltpu.VMEM((rows_per_chunk,), jnp.int32),
          pltpu.VMEM((rows_per_chunk, D_OUT_PAD), jnp.float32),
          pltpu.VMEM((R_CHUNK, D_OUT_PAD), jnp.float32),
          pltpu.SemaphoreType.DMA,
      ],
      compiler_params=pltpu.CompilerParams(use_tc_tiling_on_sc=False),
  )
  def k(table_hbm, idx_hbm, out_hbm, idx_v, rows_v, acc_v, gsem):
    wid = lax.axis_index("s") * NC + lax.axis_index("c")
    nbase = wid * R_PERW

    def chunk_body(j, carry):
      n0 = nbase + j * R_CHUNK
      pltpu.sync_copy(
          idx_hbm.at[pl.ds(n0 * DEG, rows_per_chunk)], idx_v)
      pltpu.async_copy(table_hbm.at[idx_v], rows_v, gsem).wait()
      for i in range(R_CHUNK):
        r0 = i * DEG
        for seg in range(D_OUT_PAD // 16):
          cs = pl.ds(seg * 16, 16)
          acc = rows_v[r0, cs]
          for t in range(1, DEG):
            acc = acc + rows_v[r0 + t, cs]
          acc_v[i, cs] = acc
      pltpu.sync_copy(acc_v, out_hbm.at[pl.ds(n0, R_CHUNK)])
      return carry

    lax.fori_loop(0, R_NCH, chunk_body, 0)

  return k(table, idx)


def _sigmoid(x):
  return 0.5 * jnp.tanh(0.5 * x) + 0.5


def _sage_lstm_layer(x, mb_t, WihT, WhhT, b_gate, WselfT, WneighT, b_out,
                     relu, d_out, out_dtype=jnp.float32):
  B = 1000
  n_blk = N // B

  def body(x_ref, mb_ref, wih_ref, whh_ref, bg_ref, ws_ref, wn_ref, bo_ref,
           out_ref):
    wih = wih_ref[...].astype(jnp.bfloat16)
    whh = whh_ref[...].astype(jnp.bfloat16)
    bg = bg_ref[0][None, :]

    def step(t, hc):
      h, c = hc
      xt = mb_ref[t].astype(jnp.bfloat16)
      g = (jnp.dot(xt, wih, preferred_element_type=jnp.float32)
           + jnp.dot(h.astype(jnp.bfloat16), whh,
                     preferred_element_type=jnp.float32) + bg)
      i = _sigmoid(g[:, 0:D])
      f = _sigmoid(g[:, D:2 * D])
      gg = jnp.tanh(g[:, 2 * D:3 * D])
      o = _sigmoid(g[:, 3 * D:4 * D])
      c = f * c + i * gg
      h = o * jnp.tanh(c)
      return (h, c)

    h0 = jnp.zeros((B, D), jnp.float32)
    h, _ = lax.fori_loop(0, DEG, step, (h0, h0))
    out = (jnp.dot(x_ref[...].astype(jnp.bfloat16),
                   ws_ref[...].astype(jnp.bfloat16),
                   preferred_element_type=jnp.float32)
           + jnp.dot(h.astype(jnp.bfloat16), wn_ref[...].astype(jnp.bfloat16),
                     preferred_element_type=jnp.float32)
           + bo_ref[0][None, :])
    if relu:
      out = jnp.maximum(out, 0.0)
    out_ref[...] = out.astype(out_dtype)

  return pl.pallas_call(
      body,
      grid=(n_blk,),
      in_specs=[
          pl.BlockSpec((B, D), lambda i: (i, 0)),
          pl.BlockSpec((DEG, B, D), lambda i: (0, i, 0)),
          pl.BlockSpec((D, 4 * D), lambda i: (0, 0)),
          pl.BlockSpec((D, 4 * D), lambda i: (0, 0)),
          pl.BlockSpec((8, 4 * D), lambda i: (0, 0)),
          pl.BlockSpec((D, d_out), lambda i: (0, 0)),
          pl.BlockSpec((D, d_out), lambda i: (0, 0)),
          pl.BlockSpec((8, d_out), lambda i: (0, 0)),
      ],
      out_specs=pl.BlockSpec((B, d_out), lambda i: (i, 0)),
      out_shape=jax.ShapeDtypeStruct((N, d_out), out_dtype),
  )(x, mb_t, WihT, WhhT, b_gate, WselfT, WneighT, b_out)


def kernel(feat, edge_index, Wih1, Whh1, bih1, bhh1, Wself1, bself1, Wneigh1,
           bneigh1, Wih2, Whh2, bih2, bhh2, Wself2, bself2, Wneigh2, bneigh2):
  src = edge_index[0]
  idx_t = src.reshape(N, DEG).T.reshape(-1)

  WihT1 = Wih1.T
  WhhT1 = Whh1.T
  bg1 = jnp.broadcast_to((bih1 + bhh1)[None, :], (8, 4 * D))
  WsT1 = Wself1.T
  WnT1 = Wneigh1.T
  bo1 = jnp.broadcast_to((bself1 + bneigh1)[None, :], (8, D))

  WihT2 = Wih2.T
  WhhT2 = Whh2.T
  bg2 = jnp.broadcast_to((bih2 + bhh2)[None, :], (8, 4 * D))
  pad = D_OUT_PAD - N_CLS
  WsT2 = jnp.pad(Wself2, ((0, pad), (0, 0))).T
  WnT2 = jnp.pad(Wneigh2, ((0, pad), (0, 0))).T
  bo2 = jnp.broadcast_to(
      jnp.pad(bself2 + bneigh2, (0, pad))[None, :], (8, D_OUT_PAD))

  def _to_i32(t):
    return lax.bitcast_convert_type(t.reshape(t.shape[0], -1, 2), jnp.int32)

  def _to_bf16(t, n_rows):
    return lax.bitcast_convert_type(t, jnp.bfloat16).reshape(n_rows, -1)

  feat_bf = feat.astype(jnp.bfloat16)
  mb1 = _gather_rows(_to_i32(feat_bf), idx_t, E, D // 2, G_CHUNK,
                     untiled=True, dtype=jnp.int32)
  mb1 = _to_bf16(mb1, E).reshape(DEG, N, D)
  h1 = _sage_lstm_layer(feat_bf, mb1, WihT1, WhhT1, bg1, WsT1, WnT1, bo1,
                        relu=True, d_out=D, out_dtype=jnp.bfloat16)

  mb2 = _gather_rows(_to_i32(h1), idx_t, E, D // 2, G_CHUNK,
                     untiled=True, dtype=jnp.int32)
  mb2 = _to_bf16(mb2, E).reshape(DEG, N, D)
  h2 = _sage_lstm_layer(h1, mb2, WihT2, WhhT2, bg2, WsT2, WnT2, bo2,
                        relu=False, d_out=D_OUT_PAD)

  h2p = jnp.pad(h2, ((0, NP - N), (0, 0)))
  src_p = jnp.pad(src, (0, EP - E))
  outp = _gather_reduce(h2p, src_p)
  return outp[:N, :N_CLS]


# speedup vs baseline: 2.6552x; 2.6552x over previous
import functools

import jax
import jax.numpy as jnp
from jax import lax
from jax.experimental import pallas as pl
from jax.experimental.pallas import tpu as pltpu
from jax.experimental.pallas import tpu_sc as plsc

N = 10000
DEG = 16
E = N * DEG
D = 128
N_CLS = 40
D_OUT_PAD = 48

NC = 2
NS = 16
NW = NC * NS

G_PERW = E // NW
G_CHUNK = 200
G_NCH = G_PERW // G_CHUNK

NP = 10240
R_PERW = NP // NW
R_CHUNK = 32
R_NCH = R_PERW // R_CHUNK
EP = NP * DEG


def _sc_mesh():
  return plsc.VectorSubcoreMesh(
      core_axis_name="c", subcore_axis_name="s",
      num_cores=NC, num_subcores=NS)


def _gather_rows(table, idx, n_rows, width, chunk, untiled=False):
  perw = n_rows // NW
  nch = perw // chunk
  params = (pltpu.CompilerParams(use_tc_tiling_on_sc=False)
            if untiled else None)

  @functools.partial(
      pl.kernel,
      out_type=jax.ShapeDtypeStruct((n_rows, width), jnp.float32),
      mesh=_sc_mesh(),
      scratch_types=[
          pltpu.VMEM((chunk,), jnp.int32),
          pltpu.VMEM((chunk,), jnp.int32),
          pltpu.VMEM((chunk, width), jnp.float32),
          pltpu.VMEM((chunk, width), jnp.float32),
          pltpu.SemaphoreType.DMA,
          pltpu.SemaphoreType.DMA,
          pltpu.SemaphoreType.DMA,
          pltpu.SemaphoreType.DMA,
      ],
      compiler_params=params,
  )
  def k(table_hbm, idx_hbm, out_hbm, idx0, idx1, rows0, rows1, g0, g1, s0, s1):
    G_CHUNK = chunk
    G_NCH = nch
    wid = lax.axis_index("s") * NC + lax.axis_index("c")
    base = wid * perw
    idx_v = (idx0, idx1)
    rows_v = (rows0, rows1)
    gsem = (g0, g1)
    ssem = (s0, s1)
    gathers = [None, None]
    scatters = [None, None]
    for j in range(G_NCH):
      sl = j & 1
      off = base + j * G_CHUNK
      if scatters[sl] is not None:
        scatters[sl].wait()
        scatters[sl] = None
      pltpu.sync_copy(idx_hbm.at[pl.ds(off, G_CHUNK)], idx_v[sl])
      gathers[sl] = pltpu.async_copy(
          table_hbm.at[idx_v[sl]], rows_v[sl], gsem[sl])
      po = 1 - sl
      if gathers[po] is not None:
        gathers[po].wait()
        poff = base + (j - 1) * G_CHUNK
        scatters[po] = pltpu.async_copy(
            rows_v[po], out_hbm.at[pl.ds(poff, G_CHUNK)], ssem[po])
        gathers[po] = None
    sl = (G_NCH - 1) & 1
    gathers[sl].wait()
    if scatters[sl] is not None:
      scatters[sl].wait()
    off = base + (G_NCH - 1) * G_CHUNK
    pltpu.sync_copy(rows_v[sl], out_hbm.at[pl.ds(off, G_CHUNK)])
    scatters[1 - sl].wait()

  return k(table, idx)


def _gather_reduce(table, idx):
  rows_per_chunk = R_CHUNK * DEG

  @functools.partial(
      pl.kernel,
      out_type=jax.ShapeDtypeStruct((NP, D_OUT_PAD), jnp.float32),
      mesh=_sc_mesh(),
      scratch_types=[
          pltpu.VMEM((rows_per_chunk,), jnp.int32),
          pltpu.VMEM((rows_per_chunk, D_OUT_PAD), jnp.float32),
          pltpu.VMEM((R_CHUNK, D_OUT_PAD), jnp.float32),
          pltpu.SemaphoreType.DMA,
      ],
      compiler_params=pltpu.CompilerParams(use_tc_tiling_on_sc=False),
  )
  def k(table_hbm, idx_hbm, out_hbm, idx_v, rows_v, acc_v, gsem):
    wid = lax.axis_index("s") * NC + lax.axis_index("c")
    nbase = wid * R_PERW

    def chunk_body(j, carry):
      n0 = nbase + j * R_CHUNK
      pltpu.sync_copy(
          idx_hbm.at[pl.ds(n0 * DEG, rows_per_chunk)], idx_v)
      pltpu.async_copy(table_hbm.at[idx_v], rows_v, gsem).wait()
      for i in range(R_CHUNK):
        r0 = i * DEG
        for seg in range(D_OUT_PAD // 16):
          cs = pl.ds(seg * 16, 16)
          acc = rows_v[r0, cs]
          for t in range(1, DEG):
            acc = acc + rows_v[r0 + t, cs]
          acc_v[i, cs] = acc
      pltpu.sync_copy(acc_v, out_hbm.at[pl.ds(n0, R_CHUNK)])
      return carry

    lax.fori_loop(0, R_NCH, chunk_body, 0)

  return k(table, idx)


def _sigmoid(x):
  return 0.5 * jnp.tanh(0.5 * x) + 0.5


def _sage_lstm_layer(x, mb_t, WihT, WhhT, b_gate, WselfT, WneighT, b_out,
                     relu, d_out):
  B = 1000
  n_blk = N // B

  def body(x_ref, mb_ref, wih_ref, whh_ref, bg_ref, ws_ref, wn_ref, bo_ref,
           out_ref):
    wih = wih_ref[...].astype(jnp.bfloat16)
    whh = whh_ref[...].astype(jnp.bfloat16)
    bg = bg_ref[0][None, :]

    def step(t, hc):
      h, c = hc
      xt = mb_ref[t].astype(jnp.bfloat16)
      g = (jnp.dot(xt, wih, preferred_element_type=jnp.float32)
           + jnp.dot(h.astype(jnp.bfloat16), whh,
                     preferred_element_type=jnp.float32) + bg)
      i = _sigmoid(g[:, 0:D])
      f = _sigmoid(g[:, D:2 * D])
      gg = jnp.tanh(g[:, 2 * D:3 * D])
      o = _sigmoid(g[:, 3 * D:4 * D])
      c = f * c + i * gg
      h = o * jnp.tanh(c)
      return (h, c)

    h0 = jnp.zeros((B, D), jnp.float32)
    h, _ = lax.fori_loop(0, DEG, step, (h0, h0))
    out = (jnp.dot(x_ref[...], ws_ref[...], preferred_element_type=jnp.float32)
           + jnp.dot(h, wn_ref[...], preferred_element_type=jnp.float32)
           + bo_ref[0][None, :])
    if relu:
      out = jnp.maximum(out, 0.0)
    out_ref[...] = out

  return pl.pallas_call(
      body,
      grid=(n_blk,),
      in_specs=[
          pl.BlockSpec((B, D), lambda i: (i, 0)),
          pl.BlockSpec((DEG, B, D), lambda i: (0, i, 0)),
          pl.BlockSpec((D, 4 * D), lambda i: (0, 0)),
          pl.BlockSpec((D, 4 * D), lambda i: (0, 0)),
          pl.BlockSpec((8, 4 * D), lambda i: (0, 0)),
          pl.BlockSpec((D, d_out), lambda i: (0, 0)),
          pl.BlockSpec((D, d_out), lambda i: (0, 0)),
          pl.BlockSpec((8, d_out), lambda i: (0, 0)),
      ],
      out_specs=pl.BlockSpec((B, d_out), lambda i: (i, 0)),
      out_shape=jax.ShapeDtypeStruct((N, d_out), jnp.float32),
  )(x, mb_t, WihT, WhhT, b_gate, WselfT, WneighT, b_out)


def kernel(feat, edge_index, Wih1, Whh1, bih1, bhh1, Wself1, bself1, Wneigh1,
           bneigh1, Wih2, Whh2, bih2, bhh2, Wself2, bself2, Wneigh2, bneigh2):
  src = edge_index[0]
  idx_t = src.reshape(N, DEG).T.reshape(-1)

  WihT1 = Wih1.T
  WhhT1 = Whh1.T
  bg1 = jnp.broadcast_to((bih1 + bhh1)[None, :], (8, 4 * D))
  WsT1 = Wself1.T
  WnT1 = Wneigh1.T
  bo1 = jnp.broadcast_to((bself1 + bneigh1)[None, :], (8, D))

  WihT2 = Wih2.T
  WhhT2 = Whh2.T
  bg2 = jnp.broadcast_to((bih2 + bhh2)[None, :], (8, 4 * D))
  pad = D_OUT_PAD - N_CLS
  WsT2 = jnp.pad(Wself2, ((0, pad), (0, 0))).T
  WnT2 = jnp.pad(Wneigh2, ((0, pad), (0, 0))).T
  bo2 = jnp.broadcast_to(
      jnp.pad(bself2 + bneigh2, (0, pad))[None, :], (8, D_OUT_PAD))

  mb1 = _gather_rows(feat, idx_t, E, D, G_CHUNK)
  mb1 = mb1.reshape(DEG, N, D)
  h1 = _sage_lstm_layer(feat, mb1, WihT1, WhhT1, bg1, WsT1, WnT1, bo1,
                        relu=True, d_out=D)

  mb2 = _gather_rows(h1, idx_t, E, D, G_CHUNK)
  mb2 = mb2.reshape(DEG, N, D)
  h2 = _sage_lstm_layer(h1, mb2, WihT2, WhhT2, bg2, WsT2, WnT2, bo2,
                        relu=False, d_out=D_OUT_PAD)

  h2p = jnp.pad(h2, ((0, NP - N), (0, 0)))
  src_p = jnp.pad(src, (0, EP - E))
  outp = _gather_reduce(h2p, src_p)
  return outp[:N, :N_CLS]
